# Initial kernel scaffold; baseline (speedup 1.0000x reference)
import jax
import jax.numpy as jnp
from jax.experimental import pallas as pl


def kernel(x, W_enc, b_enc, W_dec, b_dec):
    raise NotImplementedError("write your pallas kernel here")


# speedup vs baseline: 1.0034x; 1.0034x over previous
import jax
import jax.numpy as jnp
from jax.experimental import pallas as pl
from jax.experimental.pallas import tpu as pltpu

D_IN = 768
NUM_LATENTS = 32768
K = 32

BN = 256


def _matmul_body(x_ref, w_ref, be_ref, bd_ref, pre_ref):
    xc = x_ref[...] - bd_ref[...]
    pre_ref[...] = (
        jnp.dot(xc, w_ref[...].T, preferred_element_type=jnp.float32)
        + be_ref[...]
    )


def kernel(x, W_enc, b_enc, W_dec, b_dec):
    B = x.shape[0]
    pre = pl.pallas_call(
        _matmul_body,
        grid=(NUM_LATENTS // BN,),
        in_specs=[
            pl.BlockSpec((B, D_IN), lambda j: (0, 0)),
            pl.BlockSpec((BN, D_IN), lambda j: (j, 0)),
            pl.BlockSpec((BN,), lambda j: (j,)),
            pl.BlockSpec((D_IN,), lambda j: (0,)),
        ],
        out_specs=pl.BlockSpec((B, BN), lambda j: (0, j)),
        out_shape=jax.ShapeDtypeStruct((B, NUM_LATENTS), jnp.float32),
    )(x, W_enc, b_enc, b_dec)

    top_acts, top_idx = jax.lax.top_k(pre, K)
    selected = jnp.take(W_dec, top_idx, axis=0)
    out = jnp.sum(top_acts[..., None] * selected, axis=1) + b_dec
    return out


# speedup vs baseline: 4.2958x; 4.2813x over previous
import jax
import jax.numpy as jnp
from jax import lax
from jax.experimental import pallas as pl
from jax.experimental.pallas import tpu as pltpu

K = 32
C = 32
BN = 512
RB = 256


def _k1_body(x_ref, w_ref, be_ref, bd_ref, pre_ref, bmt_ref):
    B = x_ref.shape[0]
    xc = x_ref[...] - bd_ref[...]
    p = jnp.dot(xc, w_ref[...].T, preferred_element_type=jnp.float32) + be_ref[...]
    pre_ref[...] = p
    bmt_ref[...] = jnp.max(p.reshape(B, BN // C, C), axis=2).T


def _k2_body(bmt_ref, bids_ref, t_ref):
    G = bmt_ref.shape[0]
    v = bmt_ref[...]
    giota = lax.broadcasted_iota(jnp.int32, (G, RB), 0)
    kiota = lax.broadcasted_iota(jnp.int32, (K, RB), 0)

    def body(i, carry):
        v, bids, _ = carry
        m = jnp.max(v, axis=0, keepdims=True)
        g = jnp.min(jnp.where(v == m, giota, G), axis=0, keepdims=True)
        bids = jnp.where(kiota == i, g, bids)
        v = jnp.where(giota == g, -jnp.inf, v)
        return v, bids, m

    v, bids, m = lax.fori_loop(
        0, K, body,
        (v, jnp.zeros((K, RB), jnp.int32), jnp.zeros((1, RB), jnp.float32)))
    bids_ref[...] = bids
    t_ref[...] = m


def kernel(x, W_enc, b_enc, W_dec, b_dec):
    B, d = x.shape
    L = W_enc.shape[0]
    G = L // C

    pre, bmt = pl.pallas_call(
        _k1_body,
        grid=(L // BN,),
        in_specs=[
            pl.BlockSpec((B, d), lambda j: (0, 0)),
            pl.BlockSpec((BN, d), lambda j: (j, 0)),
            pl.BlockSpec((BN,), lambda j: (j,)),
            pl.BlockSpec((d,), lambda j: (0,)),
        ],
        out_specs=[
            pl.BlockSpec((B, BN), lambda j: (0, j)),
            pl.BlockSpec((BN // C, B), lambda j: (j, 0)),
        ],
        out_shape=[
            jax.ShapeDtypeStruct((B, L), jnp.float32),
            jax.ShapeDtypeStruct((G, B), jnp.float32),
        ],
    )(x, W_enc, b_enc, b_dec)

    bids_t, _tval = pl.pallas_call(
        _k2_body,
        grid=(B // RB,),
        in_specs=[pl.BlockSpec((G, RB), lambda i: (0, i))],
        out_specs=[
            pl.BlockSpec((K, RB), lambda i: (0, i)),
            pl.BlockSpec((1, RB), lambda i: (0, i)),
        ],
        out_shape=[
            jax.ShapeDtypeStruct((K, B), jnp.int32),
            jax.ShapeDtypeStruct((1, B), jnp.float32),
        ],
    )(bmt)
    bids = bids_t.T

    cand = jnp.take_along_axis(
        pre.reshape(B, G, C), bids[..., None], axis=1).reshape(B, K * C)
    cvals, cpos = jax.lax.top_k(cand, K)
    gidx = jnp.take_along_axis(bids, cpos // C, axis=1) * C + cpos % C
    selected = jnp.take(W_dec, gidx, axis=0)
    out = jnp.sum(cvals[..., None] * selected, axis=1) + b_dec
    return out


# speedup vs baseline: 6.0812x; 1.4156x over previous
import functools

import jax
import jax.numpy as jnp
from jax import lax
from jax.experimental import pallas as pl
from jax.experimental.pallas import tpu as pltpu
from jax.experimental.pallas import tpu_sc as plsc

K = 32
C = 32
NCAND = K * C
BN = 512
RB = 256


def _k1_body(x_ref, w_ref, be_ref, bd_ref, pre_ref, bmt_ref):
    B = x_ref.shape[0]
    xc = x_ref[...] - bd_ref[...]
    p = jnp.dot(xc, w_ref[...].T, preferred_element_type=jnp.float32) + be_ref[...]
    pre_ref[...] = p
    bmt_ref[...] = jnp.max(p.reshape(B, BN // C, C), axis=2).T


def _k2_body(bmt_ref, bids_ref, t_ref):
    G = bmt_ref.shape[0]
    v = bmt_ref[...]
    giota = lax.broadcasted_iota(jnp.int32, (G, RB), 0)
    kiota = lax.broadcasted_iota(jnp.int32, (K, RB), 0)

    def body(i, carry):
        v, bids, _ = carry
        m = jnp.max(v, axis=0, keepdims=True)
        g = jnp.min(jnp.where(v == m, giota, G), axis=0, keepdims=True)
        bids = jnp.where(kiota == i, g, bids)
        v = jnp.where(giota == g, -jnp.inf, v)
        return v, bids, m

    v, bids, m = lax.fori_loop(
        0, K, body,
        (v, jnp.zeros((K, RB), jnp.int32), jnp.zeros((1, RB), jnp.float32)))
    bids_ref[...] = bids.T
    t_ref[...] = m.T


def _sc_body(preG, bids_hbm, tval_hbm, wdec_hbm, bdec_hbm, out_hbm,
             bidv, fbv, candv, survv, survi, actsv, idxv, rowsv, bdecv,
             tvv, orow, sem):
    G_TOK = preG.shape[0] // out_hbm.shape[0]
    d = out_hbm.shape[1]
    NW = 32
    TPW = out_hbm.shape[0] // NW
    wid = lax.axis_index("s") * 2 + lax.axis_index("c")
    base = wid * TPW

    pltpu.sync_copy(tval_hbm.at[pl.ds(base, TPW)], tvv)
    pltpu.sync_copy(bdec_hbm, bdecv)
    lane = lax.iota(jnp.int32, 16)
    zero16i = jnp.zeros((16,), jnp.int32)

    def splat(ref, i):
        return plsc.load_gather(ref, [zero16i + i])

    def token_body(ti, _):
        t = base + ti
        pltpu.sync_copy(bids_hbm.at[t], bidv)
        b_lo = bidv[pl.ds(0, 16)]
        b_hi = bidv[pl.ds(16, 16)]
        fbv[pl.ds(0, 16)] = b_lo + t * G_TOK
        fbv[pl.ds(16, 16)] = b_hi + t * G_TOK
        pltpu.async_copy(preG.at[fbv], candv, sem).wait()

        T = splat(tvv, ti)

        def filt(v, cnt):
            vals = candv[v >> 1, pl.ds((v & 1) * 16, 16)]
            bsp = splat(bidv, v >> 1)
            gidx = bsp * C + (v & 1) * 16 + lane
            msk = vals >= T
            pos = cnt + plsc.cumsum(msk.astype(jnp.int32)) - 1
            plsc.store_scatter(survv, [pos], vals, mask=msk)
            plsc.store_scatter(survi, [pos], gidx, mask=msk)
            return cnt + jnp.sum(msk.astype(jnp.int32))

        cnt = lax.fori_loop(0, NCAND // 16, filt, jnp.int32(0), unroll=False)
        survv[pl.ds(cnt, 16)] = jnp.full((16,), -jnp.inf, jnp.float32)
        survi[pl.ds(cnt, 16)] = zero16i
        nv = (cnt + 15) >> 4

        def ext(i, carry):
            alo, ahi, ilo, ihi = carry

            def mx(v, acc):
                return jnp.maximum(acc, survv[pl.ds(v * 16, 16)])

            acc = lax.fori_loop(0, nv, mx, jnp.full((16,), -jnp.inf,
                                                    jnp.float32))
            m = jnp.max(acc)

            def fnd(v, pacc):
                vals = survv[pl.ds(v * 16, 16)]
                return jnp.minimum(pacc, jnp.where(vals == m, lane + v * 16,
                                                   NCAND + 16))

            p = jnp.min(lax.fori_loop(0, nv, fnd, jnp.full((16,), NCAND + 16,
                                                           jnp.int32)))
            plsc.store_scatter(survv, [zero16i + p],
                               jnp.full((16,), -jnp.inf, jnp.float32),
                               mask=lane == 0)
            g = splat(survi, p)
            msplat = jnp.zeros((16,), jnp.float32) + m
            alo = jnp.where(lane == i, msplat, alo)
            ahi = jnp.where(lane == i - 16, msplat, ahi)
            ilo = jnp.where(lane == i, g, ilo)
            ihi = jnp.where(lane == i - 16, g, ihi)
            return alo, ahi, ilo, ihi

        z16f = jnp.zeros((16,), jnp.float32)
        alo, ahi, ilo, ihi = lax.fori_loop(0, K, ext, (z16f, z16f, zero16i,
                                                       zero16i))
        actsv[pl.ds(0, 16)] = alo
        actsv[pl.ds(16, 16)] = ahi
        idxv[pl.ds(0, 16)] = ilo
        idxv[pl.ds(16, 16)] = ihi

        pltpu.async_copy(wdec_hbm.at[idxv], rowsv, sem).wait()
        for chunk in range(d // 256):
            accs = [bdecv[pl.ds(chunk * 256 + j * 16, 16)] for j in range(16)]

            def dec(k, accs):
                a = splat(actsv, k)
                return tuple(
                    accs[j] + a * rowsv[k, pl.ds(chunk * 256 + j * 16, 16)]
                    for j in range(16))

            accs = lax.fori_loop(0, K, dec, tuple(accs))
            for j in range(16):
                orow[pl.ds(chunk * 256 + j * 16, 16)] = accs[j]
        pltpu.sync_copy(orow, out_hbm.at[t])
        return 0

    lax.fori_loop(0, TPW, token_body, 0, unroll=False)


def kernel(x, W_enc, b_enc, W_dec, b_dec):
    B, d = x.shape
    L = W_enc.shape[0]
    G = L // C

    pre, bmt = pl.pallas_call(
        _k1_body,
        grid=(L // BN,),
        in_specs=[
            pl.BlockSpec((B, d), lambda j: (0, 0)),
            pl.BlockSpec((BN, d), lambda j: (j, 0)),
            pl.BlockSpec((BN,), lambda j: (j,)),
            pl.BlockSpec((d,), lambda j: (0,)),
        ],
        out_specs=[
            pl.BlockSpec((B, BN), lambda j: (0, j)),
            pl.BlockSpec((BN // C, B), lambda j: (j, 0)),
        ],
        out_shape=[
            jax.ShapeDtypeStruct((B, L), jnp.float32),
            jax.ShapeDtypeStruct((G, B), jnp.float32),
        ],
    )(x, W_enc, b_enc, b_dec)

    bids, tval = pl.pallas_call(
        _k2_body,
        grid=(B // RB,),
        in_specs=[pl.BlockSpec((G, RB), lambda i: (0, i))],
        out_specs=[
            pl.BlockSpec((RB, K), lambda i: (i, 0)),
            pl.BlockSpec((RB, 1), lambda i: (i, 0)),
        ],
        out_shape=[
            jax.ShapeDtypeStruct((B, K), jnp.int32),
            jax.ShapeDtypeStruct((B, 1), jnp.float32),
        ],
    )(bmt)

    sc = functools.partial(
        pl.kernel,
        mesh=plsc.VectorSubcoreMesh(core_axis_name="c", subcore_axis_name="s"),
        out_type=jax.ShapeDtypeStruct((B, d), jnp.float32),
        compiler_params=pltpu.CompilerParams(
            needs_layout_passes=False, use_tc_tiling_on_sc=False),
        scratch_types=[
            pltpu.VMEM((K,), jnp.int32),
            pltpu.VMEM((K,), jnp.int32),
            pltpu.VMEM((K, C), jnp.float32),
            pltpu.VMEM((NCAND + 16,), jnp.float32),
            pltpu.VMEM((NCAND + 16,), jnp.int32),
            pltpu.VMEM((K,), jnp.float32),
            pltpu.VMEM((K,), jnp.int32),
            pltpu.VMEM((K, d), jnp.float32),
            pltpu.VMEM((d,), jnp.float32),
            pltpu.VMEM((B // 32,), jnp.float32),
            pltpu.VMEM((d,), jnp.float32),
            pltpu.SemaphoreType.DMA,
        ],
    )(_sc_body)
    out = sc(pre.reshape(B * G, C), bids, tval.reshape(B), W_dec, b_dec)
    return out
